# baseline (speedup 1.0000x reference)
import functools

import jax
import jax.numpy as jnp
from jax import lax
from jax.experimental import pallas as pl
from jax.experimental.pallas import tpu as pltpu
from jax.experimental.pallas import tpu_sc as plsc

N = 50000
E = 800000
EPS = 1e-5

NPAD = 50176
B = 1568
EROWS = 6400
E_PAD = EROWS * 128
SB = 8
TR_HALF = EROWS // 32
TR_FULL = EROWS // 16
ZR = NPAD // 16
NCHUNK = 8
CS = NPAD // NCHUNK
CR = CS // 16
WB = 56

_mesh = plsc.VectorSubcoreMesh(
    core_axis_name="c", subcore_axis_name="s", num_cores=2, num_subcores=16)


def _fill_zeros_1d(ref, nwords):
    def body(i, _):
        ref[pl.ds(i * 16, 16)] = jnp.zeros((16,), jnp.float32)
        return 0
    lax.fori_loop(0, nwords // 16, body, 0)


def _fill_zeros_2d(ref, nrows):
    def body(i, _):
        for k in range(8):
            ref[i, pl.ds(k * 16, 16)] = jnp.zeros((16,), jnp.float32)
        return 0
    lax.fori_loop(0, nrows, body, 0)


@functools.partial(
    pl.kernel,
    out_type=jax.ShapeDtypeStruct((2 * NPAD,), jnp.float32),
    mesh=_mesh,
    scratch_types=[
        pltpu.VMEM((SB, 128), jnp.int32),
        pltpu.VMEM((128,), jnp.float32),
        pltpu.VMEM((ZR,), jnp.float32),
        pltpu.VMEM_SHARED((NPAD,), jnp.float32),
    ],
)
def _sc_deg(dst2d, degp, dstv, onesv, zbuf, acc):
    cid = lax.axis_index("c")
    sid = lax.axis_index("s")
    for k in range(8):
        onesv[pl.ds(k * 16, 16)] = jnp.ones((16,), jnp.float32)
    _fill_zeros_1d(zbuf, ZR)
    pltpu.sync_copy(zbuf, acc.at[pl.ds(sid * ZR, ZR)])
    plsc.subcore_barrier()
    r0 = cid * (EROWS // 2) + sid * TR_HALF

    def jb_body(jb, _):
        pltpu.sync_copy(dst2d.at[pl.ds(r0 + jb * SB, SB)], dstv)
        for j in range(SB):
            pltpu.sync_copy(onesv, acc.at[dstv.at[j]], add=True)
        return 0

    lax.fori_loop(0, TR_HALF // SB, jb_body, 0)
    plsc.subcore_barrier()
    pltpu.sync_copy(acc.at[pl.ds(sid * ZR, ZR)], zbuf)
    pltpu.sync_copy(zbuf, degp.at[pl.ds(cid * NPAD + sid * ZR, ZR)])


@functools.partial(
    pl.kernel,
    out_type=jax.ShapeDtypeStruct((2 * NPAD,), jnp.float32),
    mesh=_mesh,
    scratch_types=[
        pltpu.VMEM((SB, 128), jnp.int32),
        pltpu.VMEM((SB, 128), jnp.int32),
        pltpu.VMEM((128,), jnp.float32),
        pltpu.VMEM((ZR,), jnp.float32),
        pltpu.VMEM_SHARED((NPAD,), jnp.float32),
    ],
)
def _sc_c(dis, src2d, dst2d, cp, srcv, dstv, valbuf, zbuf, acc):
    cid = lax.axis_index("c")
    sid = lax.axis_index("s")
    _fill_zeros_1d(zbuf, ZR)
    pltpu.sync_copy(zbuf, acc.at[pl.ds(sid * ZR, ZR)])
    plsc.subcore_barrier()
    r0 = cid * (EROWS // 2) + sid * TR_HALF

    def jb_body(jb, _):
        pltpu.sync_copy(src2d.at[pl.ds(r0 + jb * SB, SB)], srcv)
        pltpu.sync_copy(dst2d.at[pl.ds(r0 + jb * SB, SB)], dstv)
        for j in range(SB):
            pltpu.sync_copy(dis.at[dstv.at[j]], valbuf)
            pltpu.sync_copy(valbuf, acc.at[srcv.at[j]], add=True)
        return 0

    lax.fori_loop(0, TR_HALF // SB, jb_body, 0)
    plsc.subcore_barrier()
    pltpu.sync_copy(acc.at[pl.ds(sid * ZR, ZR)], zbuf)
    pltpu.sync_copy(zbuf, cp.at[pl.ds(cid * NPAD + sid * ZR, ZR)])


@functools.partial(
    pl.kernel,
    out_type=jax.ShapeDtypeStruct((NPAD, 128), jnp.float32),
    mesh=_mesh,
    scratch_types=[
        pltpu.VMEM((SB, 128), jnp.int32),
        pltpu.VMEM((SB, 128), jnp.int32),
        pltpu.VMEM((SB, 128), jnp.int32),
        pltpu.VMEM((128, 128), jnp.float32),
        pltpu.VMEM((WB, 128), jnp.float32),
        pltpu.VMEM((WB, 128), jnp.float32),
        pltpu.VMEM_SHARED((CS + 8, 128), jnp.float32),
    ],
)
def _sc_prop(tbl, src2d, dst2d, out, srcv, dstv, dstl, rowbuf, zbuf, wbuf, acc):
    cid = lax.axis_index("c")
    sid = lax.axis_index("s")
    _fill_zeros_2d(zbuf, WB)
    zero16i = jnp.zeros((16,), jnp.int32)
    cs16 = jnp.full((16,), CS, jnp.int32)
    for chunk in range(NCHUNK // 2):
        cg = cid * (NCHUNK // 2) + chunk
        base = cg * CS
        base16 = jnp.full((16,), base, jnp.int32)
        for t in range(7):
            pltpu.sync_copy(zbuf, acc.at[pl.ds(sid * CR + t * WB, WB)])
        plsc.subcore_barrier()

        def jb_body(jb, _):
            pltpu.sync_copy(src2d.at[pl.ds(sid * TR_FULL + jb * SB, SB)], srcv)
            pltpu.sync_copy(dst2d.at[pl.ds(sid * TR_FULL + jb * SB, SB)], dstv)
            for j in range(SB):
                for v in range(8):
                    d = dstv[j, pl.ds(v * 16, 16)]
                    dloc = d - base16
                    m = (dloc >= zero16i) & (dloc < cs16)
                    dstl[j, pl.ds(v * 16, 16)] = jnp.where(m, dloc, cs16)
                pltpu.sync_copy(tbl.at[srcv.at[j]], rowbuf)
                pltpu.sync_copy(rowbuf, acc.at[dstl.at[j]], add=True)
            return 0

        lax.fori_loop(0, TR_FULL // SB, jb_body, 0)
        plsc.subcore_barrier()

        for t in range(7):
            pltpu.sync_copy(acc.at[pl.ds(sid * CR + t * WB, WB)], wbuf)
            pltpu.sync_copy(wbuf, out.at[pl.ds(base + sid * CR + t * WB, WB)])
        plsc.subcore_barrier()


def _layer_norm(z, g, b):
    mu = jnp.mean(z, axis=-1, keepdims=True)
    var = jnp.mean((z - mu) ** 2, axis=-1, keepdims=True)
    return (z - mu) * jax.lax.rsqrt(var + EPS) * g + b


def _dense1_body(x_ref, dis_ref, w1t_ref, y1_ref):
    y1_ref[...] = dis_ref[...] * jnp.dot(x_ref[...], w1t_ref[...],
                                         preferred_element_type=jnp.float32)


def _dense1(x, dis, w1t):
    return pl.pallas_call(
        _dense1_body,
        grid=(NPAD // B,),
        in_specs=[
            pl.BlockSpec((B, 20), lambda i: (i, 0)),
            pl.BlockSpec((B, 1), lambda i: (i, 0)),
            pl.BlockSpec((20, 128), lambda i: (0, 0)),
        ],
        out_specs=pl.BlockSpec((B, 128), lambda i: (i, 0)),
        out_shape=jax.ShapeDtypeStruct((NPAD, 128), jnp.float32),
    )(x, dis, w1t)


def _dense2_body(x_ref, y1_ref, s1_ref, dis_ref, wrt_ref, bias_ref,
                 h_ref, y2_ref):
    dis = dis_ref[...]
    h1 = dis * (s1_ref[...] + y1_ref[...])
    res = jnp.dot(x_ref[...], wrt_ref[...], preferred_element_type=jnp.float32)
    z = h1 + bias_ref[0:1, :] + res + bias_ref[1:2, :]
    h = jax.nn.relu(_layer_norm(z, bias_ref[2:3, :], bias_ref[3:4, :]))
    h_ref[...] = h
    y2_ref[...] = dis * h


def _dense2(x, y1, s1, dis, wrt, bias):
    blk = pl.BlockSpec((B, 128), lambda i: (i, 0))
    return pl.pallas_call(
        _dense2_body,
        grid=(NPAD // B,),
        in_specs=[
            pl.BlockSpec((B, 20), lambda i: (i, 0)),
            blk, blk,
            pl.BlockSpec((B, 1), lambda i: (i, 0)),
            pl.BlockSpec((20, 128), lambda i: (0, 0)),
            pl.BlockSpec((4, 128), lambda i: (0, 0)),
        ],
        out_specs=[blk, blk],
        out_shape=[jax.ShapeDtypeStruct((NPAD, 128), jnp.float32),
                   jax.ShapeDtypeStruct((NPAD, 128), jnp.float32)],
    )(x, y1, s1, dis, wrt, bias)


def _dense3_body(h_ref, y2_ref, s2_ref, dis_ref, cw_ref, w2t_ref, bias_ref,
                 w3t_ref, b3_ref, out_ref, acc_ref):
    i = pl.program_id(0)
    dis = dis_ref[...]
    agg2 = dis * (s2_ref[...] + y2_ref[...])
    t = jnp.dot(agg2, w2t_ref[...], preferred_element_type=jnp.float32)
    h2 = jax.nn.relu(_layer_norm(t + bias_ref[0:1, :] + h_ref[...],
                                 bias_ref[1:2, :], bias_ref[2:3, :]))
    row = i * B + lax.broadcasted_iota(jnp.int32, (B, 1), 0)
    w = jnp.where(row < N, dis * (cw_ref[...] + dis), 0.0)
    part = jnp.sum(w * h2, axis=0, keepdims=True)

    @pl.when(i == 0)
    def _():
        acc_ref[...] = jnp.zeros_like(acc_ref)

    acc_ref[...] += part

    @pl.when(i == pl.num_programs(0) - 1)
    def _():
        pooled = acc_ref[...] * (1.0 / N)
        out_ref[...] = jnp.dot(pooled, w3t_ref[...],
                               preferred_element_type=jnp.float32) + b3_ref[...]


def _dense3(h, y2, s2, dis, c, w2t, bias, w3t, b3):
    blk = pl.BlockSpec((B, 128), lambda i: (i, 0))
    return pl.pallas_call(
        _dense3_body,
        grid=(NPAD // B,),
        in_specs=[
            blk, blk, blk,
            pl.BlockSpec((B, 1), lambda i: (i, 0)),
            pl.BlockSpec((B, 1), lambda i: (i, 0)),
            pl.BlockSpec((128, 128), lambda i: (0, 0)),
            pl.BlockSpec((3, 128), lambda i: (0, 0)),
            pl.BlockSpec((128, 256), lambda i: (0, 0)),
            pl.BlockSpec((1, 256), lambda i: (0, 0)),
        ],
        out_specs=pl.BlockSpec((1, 256), lambda i: (0, 0)),
        out_shape=jax.ShapeDtypeStruct((1, 256), jnp.float32),
        scratch_shapes=[pltpu.VMEM((1, 128), jnp.float32)],
    )(h, y2, s2, dis, c, w2t, bias, w3t, b3)


def kernel(x, edge_index, W1, b1, W2, b2, W3, b3, Wr, br, g1, be1, g2, be2):
    src = edge_index[0]
    dst = edge_index[1]
    npad_e = E_PAD - E
    srcp = jnp.concatenate([src, jnp.zeros((npad_e,), jnp.int32)])
    dstp = jnp.concatenate([dst, jnp.full((npad_e,), N, jnp.int32)])
    src2d = srcp.reshape(EROWS, 128)
    dst2d = dstp.reshape(EROWS, 128)
    xp = jnp.pad(x, ((0, NPAD - N), (0, 0)))

    degp = _sc_deg(dst2d)
    deg = 1.0 + degp[:NPAD] + degp[NPAD:]
    dis = jax.lax.rsqrt(deg)
    dis2d = dis[:, None]

    cp = _sc_c(dis, src2d, dst2d)
    c2d = (cp[:NPAD] + cp[NPAD:])[:, None]

    y1 = _dense1(xp, dis2d, W1.T)
    s1 = _sc_prop(y1, src2d, dst2d)

    bias2 = jnp.stack([b1, br, g1, be1])
    h, y2 = _dense2(xp, y1, s1, dis2d, Wr.T, bias2)

    s2 = _sc_prop(y2, src2d, dst2d)

    bias3 = jnp.stack([b2, g2, be2])
    out = _dense3(h, y2, s2, dis2d, c2d, W2.T, bias3, W3.T, b3[None, :])
    return out


# speedup vs baseline: 1.9490x; 1.9490x over previous
import functools

import jax
import jax.numpy as jnp
from jax import lax
from jax.experimental import pallas as pl
from jax.experimental.pallas import tpu as pltpu
from jax.experimental.pallas import tpu_sc as plsc

N = 50000
E = 800000
EPS = 1e-5

NPAD = 50176
B = 1568
EROWS = 6400
E_PAD = EROWS * 128
SB = 8
TR_HALF = EROWS // 32
TR_FULL = EROWS // 16
ZR = NPAD // 16
NCHUNK = 4
CS = NPAD // NCHUNK
CR = CS // 16
WB = 16

_mesh = plsc.VectorSubcoreMesh(
    core_axis_name="c", subcore_axis_name="s", num_cores=2, num_subcores=16)


def _fill_zeros_1d(ref, nwords):
    def body(i, _):
        ref[pl.ds(i * 16, 16)] = jnp.zeros((16,), jnp.float32)
        return 0
    lax.fori_loop(0, nwords // 16, body, 0)


def _fill_zeros_2d(ref, nrows):
    def body(i, _):
        for k in range(8):
            ref[i, pl.ds(k * 16, 16)] = jnp.zeros((16,), jnp.float32)
        return 0
    lax.fori_loop(0, nrows, body, 0)


@functools.partial(
    pl.kernel,
    out_type=jax.ShapeDtypeStruct((2 * NPAD,), jnp.float32),
    mesh=_mesh,
    scratch_types=[
        pltpu.VMEM((SB, 128), jnp.int32),
        pltpu.VMEM((128,), jnp.float32),
        pltpu.VMEM((ZR,), jnp.float32),
        pltpu.VMEM_SHARED((NPAD,), jnp.float32),
    ],
)
def _sc_deg(dst2d, degp, dstv, onesv, zbuf, acc):
    cid = lax.axis_index("c")
    sid = lax.axis_index("s")
    for k in range(8):
        onesv[pl.ds(k * 16, 16)] = jnp.ones((16,), jnp.float32)
    _fill_zeros_1d(zbuf, ZR)
    pltpu.sync_copy(zbuf, acc.at[pl.ds(sid * ZR, ZR)])
    plsc.subcore_barrier()
    r0 = cid * (EROWS // 2) + sid * TR_HALF

    def jb_body(jb, _):
        pltpu.sync_copy(dst2d.at[pl.ds(r0 + jb * SB, SB)], dstv)
        for j in range(SB):
            pltpu.sync_copy(onesv, acc.at[dstv.at[j]], add=True)
        return 0

    lax.fori_loop(0, TR_HALF // SB, jb_body, 0)
    plsc.subcore_barrier()
    pltpu.sync_copy(acc.at[pl.ds(sid * ZR, ZR)], zbuf)
    pltpu.sync_copy(zbuf, degp.at[pl.ds(cid * NPAD + sid * ZR, ZR)])


@functools.partial(
    pl.kernel,
    out_type=jax.ShapeDtypeStruct((2 * NPAD,), jnp.float32),
    mesh=_mesh,
    scratch_types=[
        pltpu.VMEM((SB, 128), jnp.int32),
        pltpu.VMEM((SB, 128), jnp.int32),
        pltpu.VMEM((128,), jnp.float32),
        pltpu.VMEM((ZR,), jnp.float32),
        pltpu.VMEM_SHARED((NPAD,), jnp.float32),
    ],
)
def _sc_c(dis, src2d, dst2d, cp, srcv, dstv, valbuf, zbuf, acc):
    cid = lax.axis_index("c")
    sid = lax.axis_index("s")
    _fill_zeros_1d(zbuf, ZR)
    pltpu.sync_copy(zbuf, acc.at[pl.ds(sid * ZR, ZR)])
    plsc.subcore_barrier()
    r0 = cid * (EROWS // 2) + sid * TR_HALF

    def jb_body(jb, _):
        pltpu.sync_copy(src2d.at[pl.ds(r0 + jb * SB, SB)], srcv)
        pltpu.sync_copy(dst2d.at[pl.ds(r0 + jb * SB, SB)], dstv)
        for j in range(SB):
            pltpu.sync_copy(dis.at[dstv.at[j]], valbuf)
            pltpu.sync_copy(valbuf, acc.at[srcv.at[j]], add=True)
        return 0

    lax.fori_loop(0, TR_HALF // SB, jb_body, 0)
    plsc.subcore_barrier()
    pltpu.sync_copy(acc.at[pl.ds(sid * ZR, ZR)], zbuf)
    pltpu.sync_copy(zbuf, cp.at[pl.ds(cid * NPAD + sid * ZR, ZR)])


@functools.partial(
    pl.kernel,
    out_type=jax.ShapeDtypeStruct((NPAD, 128), jnp.float32),
    mesh=_mesh,
    scratch_types=[
        pltpu.VMEM((SB, 128), jnp.int32),
        pltpu.VMEM((SB, 128), jnp.int32),
        pltpu.VMEM((SB, 128), jnp.int32),
        pltpu.VMEM((128, 128), jnp.float32),
        pltpu.VMEM((WB, 128), jnp.float32),
        pltpu.VMEM((WB, 128), jnp.float32),
        pltpu.VMEM_SHARED((CS + 8, 128), jnp.float32),
    ],
)
def _sc_prop(tbl, src2d, dst2d, out, srcv, dstv, dstl, rowbuf, zbuf, wbuf, acc):
    cid = lax.axis_index("c")
    sid = lax.axis_index("s")
    _fill_zeros_2d(zbuf, WB)
    zero16i = jnp.zeros((16,), jnp.int32)
    cs16 = jnp.full((16,), CS, jnp.int32)
    for chunk in range(NCHUNK // 2):
        cg = cid * (NCHUNK // 2) + chunk
        base = cg * CS
        base16 = jnp.full((16,), base, jnp.int32)
        for t in range(CR // WB):
            pltpu.sync_copy(zbuf, acc.at[pl.ds(sid * CR + t * WB, WB)])
        plsc.subcore_barrier()

        def jb_body(jb, _):
            pltpu.sync_copy(src2d.at[pl.ds(sid * TR_FULL + jb * SB, SB)], srcv)
            pltpu.sync_copy(dst2d.at[pl.ds(sid * TR_FULL + jb * SB, SB)], dstv)
            for j in range(SB):
                for v in range(8):
                    d = dstv[j, pl.ds(v * 16, 16)]
                    dloc = d - base16
                    m = (dloc >= zero16i) & (dloc < cs16)
                    dstl[j, pl.ds(v * 16, 16)] = jnp.where(m, dloc, cs16)
                pltpu.sync_copy(tbl.at[srcv.at[j]], rowbuf)
                pltpu.sync_copy(rowbuf, acc.at[dstl.at[j]], add=True)
            return 0

        lax.fori_loop(0, TR_FULL // SB, jb_body, 0)
        plsc.subcore_barrier()

        for t in range(CR // WB):
            pltpu.sync_copy(acc.at[pl.ds(sid * CR + t * WB, WB)], wbuf)
            pltpu.sync_copy(wbuf, out.at[pl.ds(base + sid * CR + t * WB, WB)])
        plsc.subcore_barrier()


def _layer_norm(z, g, b):
    mu = jnp.mean(z, axis=-1, keepdims=True)
    var = jnp.mean((z - mu) ** 2, axis=-1, keepdims=True)
    return (z - mu) * jax.lax.rsqrt(var + EPS) * g + b


def _dense1_body(x_ref, dis_ref, w1t_ref, y1_ref):
    y1_ref[...] = dis_ref[...] * jnp.dot(x_ref[...], w1t_ref[...],
                                         preferred_element_type=jnp.float32)


def _dense1(x, dis, w1t):
    return pl.pallas_call(
        _dense1_body,
        grid=(NPAD // B,),
        in_specs=[
            pl.BlockSpec((B, 20), lambda i: (i, 0)),
            pl.BlockSpec((B, 1), lambda i: (i, 0)),
            pl.BlockSpec((20, 128), lambda i: (0, 0)),
        ],
        out_specs=pl.BlockSpec((B, 128), lambda i: (i, 0)),
        out_shape=jax.ShapeDtypeStruct((NPAD, 128), jnp.float32),
    )(x, dis, w1t)


def _dense2_body(x_ref, y1_ref, s1_ref, dis_ref, wrt_ref, bias_ref,
                 h_ref, y2_ref):
    dis = dis_ref[...]
    h1 = dis * (s1_ref[...] + y1_ref[...])
    res = jnp.dot(x_ref[...], wrt_ref[...], preferred_element_type=jnp.float32)
    z = h1 + bias_ref[0:1, :] + res + bias_ref[1:2, :]
    h = jax.nn.relu(_layer_norm(z, bias_ref[2:3, :], bias_ref[3:4, :]))
    h_ref[...] = h
    y2_ref[...] = dis * h


def _dense2(x, y1, s1, dis, wrt, bias):
    blk = pl.BlockSpec((B, 128), lambda i: (i, 0))
    return pl.pallas_call(
        _dense2_body,
        grid=(NPAD // B,),
        in_specs=[
            pl.BlockSpec((B, 20), lambda i: (i, 0)),
            blk, blk,
            pl.BlockSpec((B, 1), lambda i: (i, 0)),
            pl.BlockSpec((20, 128), lambda i: (0, 0)),
            pl.BlockSpec((4, 128), lambda i: (0, 0)),
        ],
        out_specs=[blk, blk],
        out_shape=[jax.ShapeDtypeStruct((NPAD, 128), jnp.float32),
                   jax.ShapeDtypeStruct((NPAD, 128), jnp.float32)],
    )(x, y1, s1, dis, wrt, bias)


def _dense3_body(h_ref, y2_ref, s2_ref, dis_ref, cw_ref, w2t_ref, bias_ref,
                 w3t_ref, b3_ref, out_ref, acc_ref):
    i = pl.program_id(0)
    dis = dis_ref[...]
    agg2 = dis * (s2_ref[...] + y2_ref[...])
    t = jnp.dot(agg2, w2t_ref[...], preferred_element_type=jnp.float32)
    h2 = jax.nn.relu(_layer_norm(t + bias_ref[0:1, :] + h_ref[...],
                                 bias_ref[1:2, :], bias_ref[2:3, :]))
    row = i * B + lax.broadcasted_iota(jnp.int32, (B, 1), 0)
    w = jnp.where(row < N, dis * (cw_ref[...] + dis), 0.0)
    part = jnp.sum(w * h2, axis=0, keepdims=True)

    @pl.when(i == 0)
    def _():
        acc_ref[...] = jnp.zeros_like(acc_ref)

    acc_ref[...] += part

    @pl.when(i == pl.num_programs(0) - 1)
    def _():
        pooled = acc_ref[...] * (1.0 / N)
        out_ref[...] = jnp.dot(pooled, w3t_ref[...],
                               preferred_element_type=jnp.float32) + b3_ref[...]


def _dense3(h, y2, s2, dis, c, w2t, bias, w3t, b3):
    blk = pl.BlockSpec((B, 128), lambda i: (i, 0))
    return pl.pallas_call(
        _dense3_body,
        grid=(NPAD // B,),
        in_specs=[
            blk, blk, blk,
            pl.BlockSpec((B, 1), lambda i: (i, 0)),
            pl.BlockSpec((B, 1), lambda i: (i, 0)),
            pl.BlockSpec((128, 128), lambda i: (0, 0)),
            pl.BlockSpec((3, 128), lambda i: (0, 0)),
            pl.BlockSpec((128, 256), lambda i: (0, 0)),
            pl.BlockSpec((1, 256), lambda i: (0, 0)),
        ],
        out_specs=pl.BlockSpec((1, 256), lambda i: (0, 0)),
        out_shape=jax.ShapeDtypeStruct((1, 256), jnp.float32),
        scratch_shapes=[pltpu.VMEM((1, 128), jnp.float32)],
    )(h, y2, s2, dis, c, w2t, bias, w3t, b3)


def kernel(x, edge_index, W1, b1, W2, b2, W3, b3, Wr, br, g1, be1, g2, be2):
    src = edge_index[0]
    dst = edge_index[1]
    npad_e = E_PAD - E
    srcp = jnp.concatenate([src, jnp.zeros((npad_e,), jnp.int32)])
    dstp = jnp.concatenate([dst, jnp.full((npad_e,), N, jnp.int32)])
    src2d = srcp.reshape(EROWS, 128)
    dst2d = dstp.reshape(EROWS, 128)
    xp = jnp.pad(x, ((0, NPAD - N), (0, 0)))

    degp = _sc_deg(dst2d)
    deg = 1.0 + degp[:NPAD] + degp[NPAD:]
    dis = jax.lax.rsqrt(deg)
    dis2d = dis[:, None]

    cp = _sc_c(dis, src2d, dst2d)
    c2d = (cp[:NPAD] + cp[NPAD:])[:, None]

    y1 = _dense1(xp, dis2d, W1.T)
    s1 = _sc_prop(y1, src2d, dst2d)

    bias2 = jnp.stack([b1, br, g1, be1])
    h, y2 = _dense2(xp, y1, s1, dis2d, Wr.T, bias2)

    s2 = _sc_prop(y2, src2d, dst2d)

    bias3 = jnp.stack([b2, g2, be2])
    out = _dense3(h, y2, s2, dis2d, c2d, W2.T, bias3, W3.T, b3[None, :])
    return out
